# Initial kernel scaffold; baseline (speedup 1.0000x reference)
import jax
import jax.numpy as jnp
from jax.experimental import pallas as pl


def kernel(x, b_pre, W_enc, W_dec, b_post):
    raise NotImplementedError("write your pallas kernel here")


# speedup vs baseline: 1.1686x; 1.1686x over previous
import functools
import math

import jax
import jax.numpy as jnp
from jax import lax
from jax.experimental import pallas as pl
from jax.experimental.pallas import tpu as pltpu
from jax.experimental.pallas import tpu_sc as plsc

_D = 2048
_N = 32768
_K = 32
_B = 2048
_L = 1024
_FOLDS = _N // _L
_SQRT_D = math.sqrt(_D)
_AVG_NORM = 1.0
_AUX_COEFF = 1.0 / 32.0

_ROW_BLK = 512
_NRB = _B // _ROW_BLK
_FB = 4096
_NFB = _N // _FB


def _encode_body(x_ref, w_ref, bv_ref, ba_ref):
    acc = jnp.dot(x_ref[...], w_ref[...], preferred_element_type=jnp.float32)
    bvb = acc[:, 0:128]
    bab = jnp.zeros((_ROW_BLK, 128), jnp.int32)
    for m in range(1, _FB // 128):
        sl = acc[:, m * 128:(m + 1) * 128]
        upd = sl >= bvb
        bvb = jnp.where(upd, sl, bvb)
        bab = jnp.where(upd, m, bab)
    bv_ref[...] = bvb
    ba_ref[...] = bab


def _encode(xcb, Wb):
    return pl.pallas_call(
        _encode_body,
        grid=(_NRB, _NFB),
        in_specs=[
            pl.BlockSpec((_ROW_BLK, _D), lambda r, t: (r, 0)),
            pl.BlockSpec((_D, _FB), lambda r, t: (0, t)),
        ],
        out_specs=[
            pl.BlockSpec((_ROW_BLK, 128), lambda r, t: (r, t)),
            pl.BlockSpec((_ROW_BLK, 128), lambda r, t: (r, t)),
        ],
        out_shape=[
            jax.ShapeDtypeStruct((_B, _L), jnp.float32),
            jax.ShapeDtypeStruct((_B, _L), jnp.int32),
        ],
        compiler_params=pltpu.CompilerParams(
            dimension_semantics=("parallel", "arbitrary"),
        ),
    )(xcb, Wb)


_TK_BLK = 256


def _topk_body(bv_ref, ba_ref, w_ref, i_ref):
    work = bv_ref[...]
    lane = lax.broadcasted_iota(jnp.int32, (_TK_BLK, _L), 1)
    full_idx = (lane // 128) * _FB + (lane % 128) + ba_ref[...] * 128
    wcols = []
    icols = []
    neg_inf = jnp.float32(-jnp.inf)
    for _ in range(_K):
        m = jnp.max(work, axis=1, keepdims=True)
        at_max = work == m
        jm = jnp.min(jnp.where(at_max, lane, _L), axis=1, keepdims=True)
        sel = lane == jm
        idx = jnp.sum(jnp.where(sel, full_idx, 0), axis=1, keepdims=True)
        wcols.append(m)
        icols.append(idx)
        work = jnp.where(sel, neg_inf, work)
    w_ref[...] = jnp.concatenate(wcols, axis=1)
    i_ref[...] = jnp.concatenate(icols, axis=1)


def _topk(bv, ba):
    return pl.pallas_call(
        _topk_body,
        grid=(_B // _TK_BLK,),
        in_specs=[
            pl.BlockSpec((_TK_BLK, _L), lambda r: (r, 0)),
            pl.BlockSpec((_TK_BLK, _L), lambda r: (r, 0)),
        ],
        out_specs=[
            pl.BlockSpec((_TK_BLK, _K), lambda r: (r, 0)),
            pl.BlockSpec((_TK_BLK, _K), lambda r: (r, 0)),
        ],
        out_shape=[
            jax.ShapeDtypeStruct((_B, _K), jnp.float32),
            jax.ShapeDtypeStruct((_B, _K), jnp.int32),
        ],
        compiler_params=pltpu.CompilerParams(
            dimension_semantics=("parallel",),
        ),
    )(bv, ba)


_NW = 32
_RPW = _B // _NW
_GROWS = 16


def _decode_body(wdec_hbm, w_hbm, idx_hbm, out_hbm, idx_v, w_v, rows_v, acc_v,
                 gsem, osem):
    wid = lax.axis_index("c") * 16 + lax.axis_index("s")
    base = wid * _RPW
    pltpu.sync_copy(idx_hbm.at[pl.ds(base, _RPW)], idx_v)
    pltpu.sync_copy(w_hbm.at[pl.ds(base * _K, _RPW * _K)], w_v)

    def row_loop(r, carry):
        for h in range(_K // _GROWS):
            pltpu.async_copy(
                wdec_hbm.at[idx_v.at[r, pl.ds(h * _GROWS, _GROWS)]],
                rows_v, gsem).wait()
            wvec = w_v[pl.ds(r * _K + h * _GROWS, _GROWS)]

            def chunk_loop(c, carry, h=h, wvec=wvec):
                off = c * 16
                if h == 0:
                    a = jnp.zeros((16,), jnp.float32)
                else:
                    a = acc_v[pl.ds(off, 16)]
                for j in range(_GROWS):
                    a = a + jnp.full((16,), wvec[j], jnp.float32) * \
                        rows_v[j, pl.ds(off, 16)]
                acc_v[pl.ds(off, 16)] = a
                return carry

            carry = lax.fori_loop(0, _D // 16, chunk_loop, carry)

        pltpu.sync_copy(acc_v, out_hbm.at[base + r])
        return carry

    lax.fori_loop(0, _RPW, row_loop, 0)


def _decode(W_dec, weights, indices):
    mesh = plsc.VectorSubcoreMesh(core_axis_name="c", subcore_axis_name="s")
    run = pl.kernel(
        _decode_body,
        out_type=jax.ShapeDtypeStruct((_B, _D), jnp.float32),
        mesh=mesh,
        scratch_types=[
            pltpu.VMEM((_RPW, _K), jnp.int32),
            pltpu.VMEM((_RPW * _K,), jnp.float32),
            pltpu.VMEM((_GROWS, _D), jnp.float32),
            pltpu.VMEM((_D,), jnp.float32),
            pltpu.SemaphoreType.DMA,
            pltpu.SemaphoreType.DMA,
        ],
    )
    return run(W_dec, weights.reshape(-1), indices)


_EP_BLK = 256


def _epilogue_body(x_ref, dec_ref, bp_ref, xn_ref, yn_ref, y_ref, rl_ref,
                   ls_ref):
    xn = x_ref[...] / _AVG_NORM * _SQRT_D
    yn = dec_ref[...] + bp_ref[...]
    xn_ref[...] = xn
    yn_ref[...] = yn
    y_ref[...] = yn * _AVG_NORM / _SQRT_D
    d = xn - yn
    rl = jnp.mean(d * d, axis=1)
    rl_ref[...] = rl
    ls_ref[...] = rl * (1.0 + _AUX_COEFF)


def _epilogue(x, decoded, b_post):
    return pl.pallas_call(
        _epilogue_body,
        grid=(_B // _EP_BLK,),
        in_specs=[
            pl.BlockSpec((_EP_BLK, _D), lambda r: (r, 0)),
            pl.BlockSpec((_EP_BLK, _D), lambda r: (r, 0)),
            pl.BlockSpec((1, _D), lambda r: (0, 0)),
        ],
        out_specs=[
            pl.BlockSpec((_EP_BLK, _D), lambda r: (r, 0)),
            pl.BlockSpec((_EP_BLK, _D), lambda r: (r, 0)),
            pl.BlockSpec((_EP_BLK, _D), lambda r: (r, 0)),
            pl.BlockSpec((_EP_BLK,), lambda r: (r,)),
            pl.BlockSpec((_EP_BLK,), lambda r: (r,)),
        ],
        out_shape=[
            jax.ShapeDtypeStruct((_B, _D), jnp.float32),
            jax.ShapeDtypeStruct((_B, _D), jnp.float32),
            jax.ShapeDtypeStruct((_B, _D), jnp.float32),
            jax.ShapeDtypeStruct((_B,), jnp.float32),
            jax.ShapeDtypeStruct((_B,), jnp.float32),
        ],
        compiler_params=pltpu.CompilerParams(
            dimension_semantics=("parallel",),
        ),
    )(x, decoded, b_post.reshape(1, _D))


def kernel(x, b_pre, W_enc, W_dec, b_post):
    xcb = (x / _AVG_NORM * _SQRT_D - b_pre[None, :]).astype(jnp.bfloat16)
    Wb = W_enc.astype(jnp.bfloat16)
    bv, ba = _encode(xcb, Wb)
    weights, indices = _topk(bv, ba)
    decoded = _decode(W_dec, weights, indices)
    x_normed, y_normed, y, recon_loss, loss = _epilogue(x, decoded, b_post)
    return (x_normed, x, weights, indices, y_normed, y, recon_loss,
            recon_loss, loss)


# speedup vs baseline: 1.3526x; 1.1575x over previous
import functools
import math

import jax
import jax.numpy as jnp
from jax import lax
from jax.experimental import pallas as pl
from jax.experimental.pallas import tpu as pltpu
from jax.experimental.pallas import tpu_sc as plsc

_D = 2048
_N = 32768
_K = 32
_B = 2048
_L = 1024
_FOLDS = _N // _L
_SQRT_D = math.sqrt(_D)
_AVG_NORM = 1.0
_AUX_COEFF = 1.0 / 32.0

_ROW_BLK = 512
_NRB = _B // _ROW_BLK
_FB = 4096
_NFB = _N // _FB


def _encode_body(x_ref, w_ref, bv_ref, ba_ref):
    acc = jnp.dot(x_ref[...], w_ref[...], preferred_element_type=jnp.float32)
    bvb = acc[:, 0:128]
    bab = jnp.zeros((_ROW_BLK, 128), jnp.int32)
    for m in range(1, _FB // 128):
        sl = acc[:, m * 128:(m + 1) * 128]
        upd = sl >= bvb
        bvb = jnp.where(upd, sl, bvb)
        bab = jnp.where(upd, m, bab)
    bv_ref[...] = bvb
    ba_ref[...] = bab


def _encode(xcb, Wb):
    return pl.pallas_call(
        _encode_body,
        grid=(_NRB, _NFB),
        in_specs=[
            pl.BlockSpec((_ROW_BLK, _D), lambda r, t: (r, 0)),
            pl.BlockSpec((_D, _FB), lambda r, t: (0, t)),
        ],
        out_specs=[
            pl.BlockSpec((_ROW_BLK, 128), lambda r, t: (r, t)),
            pl.BlockSpec((_ROW_BLK, 128), lambda r, t: (r, t)),
        ],
        out_shape=[
            jax.ShapeDtypeStruct((_B, _L), jnp.float32),
            jax.ShapeDtypeStruct((_B, _L), jnp.int32),
        ],
        compiler_params=pltpu.CompilerParams(
            dimension_semantics=("parallel", "arbitrary"),
        ),
    )(xcb, Wb)


_TK_BLK = 256


def _topk_body(bv_ref, ba_ref, w_ref, i_ref):
    work = bv_ref[...]
    lane = lax.broadcasted_iota(jnp.int32, (_TK_BLK, _L), 1)
    full_idx = (lane // 128) * _FB + (lane % 128) + ba_ref[...] * 128
    wcols = []
    icols = []
    neg_inf = jnp.float32(-jnp.inf)
    for _ in range(_K):
        m = jnp.max(work, axis=1, keepdims=True)
        at_max = work == m
        jm = jnp.min(jnp.where(at_max, lane, _L), axis=1, keepdims=True)
        sel = lane == jm
        idx = jnp.sum(jnp.where(sel, full_idx, 0), axis=1, keepdims=True)
        wcols.append(m)
        icols.append(idx)
        work = jnp.where(sel, neg_inf, work)
    w_ref[...] = jnp.concatenate(wcols, axis=1)
    i_ref[...] = jnp.concatenate(icols, axis=1)


def _topk(bv, ba):
    return pl.pallas_call(
        _topk_body,
        grid=(_B // _TK_BLK,),
        in_specs=[
            pl.BlockSpec((_TK_BLK, _L), lambda r: (r, 0)),
            pl.BlockSpec((_TK_BLK, _L), lambda r: (r, 0)),
        ],
        out_specs=[
            pl.BlockSpec((_TK_BLK, _K), lambda r: (r, 0)),
            pl.BlockSpec((_TK_BLK, _K), lambda r: (r, 0)),
        ],
        out_shape=[
            jax.ShapeDtypeStruct((_B, _K), jnp.float32),
            jax.ShapeDtypeStruct((_B, _K), jnp.int32),
        ],
        compiler_params=pltpu.CompilerParams(
            dimension_semantics=("parallel",),
        ),
    )(bv, ba)


_NW = 32
_RPW = _B // _NW
_GR = 8
_NBUF = 4


def _decode_body(wdec_hbm, w_hbm, idx_hbm, out_hbm, idx_v, w_v, rows_v, acc_v,
                 g0sem, g1sem, g2sem, g3sem, o0sem, o1sem):
    wid = lax.axis_index("c") * 16 + lax.axis_index("s")
    base = wid * _RPW
    pltpu.sync_copy(idx_hbm.at[pl.ds(base, _RPW)], idx_v)
    pltpu.sync_copy(w_hbm.at[pl.ds(base * _K, _RPW * _K)], w_v)
    gsems = (g0sem, g1sem, g2sem, g3sem)
    osems = (o0sem, o1sem)

    def gcopy(r, q, buf):
        return pltpu.make_async_copy(
            wdec_hbm.at[idx_v.at[r, pl.ds(q * _GR, _GR)]],
            rows_v.at[buf], gsems[buf])

    for q in range(4):
        gcopy(0, q, q).start()

    def pair_loop(r0, carry):
        for rr in range(2):
            r = r0 + rr
            wvA = w_v[pl.ds(r * _K, 16)]
            wvB = w_v[pl.ds(r * _K + 16, 16)]
            for q in range(4):
                buf = (rr * 4 + q) % _NBUF
                gcopy(r, q, buf).wait()
                wq = wvA if q < 2 else wvB
                bw = [jnp.full((16,), wq[(q % 2) * _GR + j], jnp.float32)
                      for j in range(_GR)]

                if q == 0:
                    @pl.when(r >= 2)
                    def _(rr=rr, r=r):
                        pltpu.make_async_copy(
                            acc_v.at[rr], out_hbm.at[base + r - 2],
                            osems[rr]).wait()

                def chunk_loop(c, carry, q=q, rr=rr, buf=buf, bw=bw):
                    off = c * 16
                    if q == 0:
                        a = jnp.zeros((16,), jnp.float32)
                    else:
                        a = acc_v[rr, pl.ds(off, 16)]
                    for j in range(_GR):
                        a = a + bw[j] * rows_v[buf, j, pl.ds(off, 16)]
                    acc_v[rr, pl.ds(off, 16)] = a
                    return carry

                carry = lax.fori_loop(0, _D // 16, chunk_loop, carry)

                @pl.when(r + 1 < _RPW)
                def _(r=r, q=q, buf=buf):
                    gcopy(r + 1, q, buf).start()

            pltpu.make_async_copy(acc_v.at[rr], out_hbm.at[base + r],
                                  osems[rr]).start()
        return carry

    lax.fori_loop(0, _RPW // 2, lambda i, c: pair_loop(i * 2, c), 0)
    pltpu.make_async_copy(acc_v.at[0], out_hbm.at[base + _RPW - 2],
                          o0sem).wait()
    pltpu.make_async_copy(acc_v.at[1], out_hbm.at[base + _RPW - 1],
                          o1sem).wait()


def _decode(W_dec, weights, indices):
    mesh = plsc.VectorSubcoreMesh(core_axis_name="c", subcore_axis_name="s")
    run = pl.kernel(
        _decode_body,
        out_type=jax.ShapeDtypeStruct((_B, _D), jnp.float32),
        mesh=mesh,
        scratch_types=[
            pltpu.VMEM((_RPW, _K), jnp.int32),
            pltpu.VMEM((_RPW * _K,), jnp.float32),
            pltpu.VMEM((_NBUF, _GR, _D), jnp.float32),
            pltpu.VMEM((2, _D), jnp.float32),
            pltpu.SemaphoreType.DMA,
            pltpu.SemaphoreType.DMA,
            pltpu.SemaphoreType.DMA,
            pltpu.SemaphoreType.DMA,
            pltpu.SemaphoreType.DMA,
            pltpu.SemaphoreType.DMA,
        ],
    )
    return run(W_dec, weights.reshape(-1), indices)


_EP_BLK = 256


def _epilogue_body(x_ref, dec_ref, bp_ref, xn_ref, yn_ref, y_ref, rl_ref,
                   ls_ref):
    xn = x_ref[...] / _AVG_NORM * _SQRT_D
    yn = dec_ref[...] + bp_ref[...]
    xn_ref[...] = xn
    yn_ref[...] = yn
    y_ref[...] = yn * _AVG_NORM / _SQRT_D
    d = xn - yn
    rl = jnp.mean(d * d, axis=1)
    rl_ref[...] = rl
    ls_ref[...] = rl * (1.0 + _AUX_COEFF)


def _epilogue(x, decoded, b_post):
    return pl.pallas_call(
        _epilogue_body,
        grid=(_B // _EP_BLK,),
        in_specs=[
            pl.BlockSpec((_EP_BLK, _D), lambda r: (r, 0)),
            pl.BlockSpec((_EP_BLK, _D), lambda r: (r, 0)),
            pl.BlockSpec((1, _D), lambda r: (0, 0)),
        ],
        out_specs=[
            pl.BlockSpec((_EP_BLK, _D), lambda r: (r, 0)),
            pl.BlockSpec((_EP_BLK, _D), lambda r: (r, 0)),
            pl.BlockSpec((_EP_BLK, _D), lambda r: (r, 0)),
            pl.BlockSpec((_EP_BLK,), lambda r: (r,)),
            pl.BlockSpec((_EP_BLK,), lambda r: (r,)),
        ],
        out_shape=[
            jax.ShapeDtypeStruct((_B, _D), jnp.float32),
            jax.ShapeDtypeStruct((_B, _D), jnp.float32),
            jax.ShapeDtypeStruct((_B, _D), jnp.float32),
            jax.ShapeDtypeStruct((_B,), jnp.float32),
            jax.ShapeDtypeStruct((_B,), jnp.float32),
        ],
        compiler_params=pltpu.CompilerParams(
            dimension_semantics=("parallel",),
        ),
    )(x, decoded, b_post.reshape(1, _D))


def kernel(x, b_pre, W_enc, W_dec, b_post):
    xcb = (x / _AVG_NORM * _SQRT_D - b_pre[None, :]).astype(jnp.bfloat16)
    Wb = W_enc.astype(jnp.bfloat16)
    bv, ba = _encode(xcb, Wb)
    weights, indices = _topk(bv, ba)
    decoded = _decode(W_dec, weights, indices)
    x_normed, y_normed, y, recon_loss, loss = _epilogue(x, decoded, b_post)
    return (x_normed, x, weights, indices, y_normed, y, recon_loss,
            recon_loss, loss)


# speedup vs baseline: 1.4695x; 1.0864x over previous
import functools
import math

import jax
import jax.numpy as jnp
from jax import lax
from jax.experimental import pallas as pl
from jax.experimental.pallas import tpu as pltpu
from jax.experimental.pallas import tpu_sc as plsc

_D = 2048
_N = 32768
_K = 32
_B = 2048
_L = 1024
_FOLDS = _N // _L
_SQRT_D = math.sqrt(_D)
_AVG_NORM = 1.0
_AUX_COEFF = 1.0 / 32.0

_ROW_BLK = 512
_NRB = _B // _ROW_BLK
_FB = 4096
_NFB = _N // _FB


def _encode_body(x_ref, w_ref, bv_ref, ba_ref):
    acc = jnp.dot(x_ref[...], w_ref[...], preferred_element_type=jnp.float32)
    bvb = acc[:, 0:128]
    bab = jnp.zeros((_ROW_BLK, 128), jnp.int32)
    for m in range(1, _FB // 128):
        sl = acc[:, m * 128:(m + 1) * 128]
        upd = sl >= bvb
        bvb = jnp.where(upd, sl, bvb)
        bab = jnp.where(upd, m, bab)
    bv_ref[...] = bvb
    ba_ref[...] = bab


def _encode(xcb, Wb):
    return pl.pallas_call(
        _encode_body,
        grid=(_NRB, _NFB),
        in_specs=[
            pl.BlockSpec((_ROW_BLK, _D), lambda r, t: (r, 0)),
            pl.BlockSpec((_D, _FB), lambda r, t: (0, t)),
        ],
        out_specs=[
            pl.BlockSpec((_ROW_BLK, 128), lambda r, t: (r, t)),
            pl.BlockSpec((_ROW_BLK, 128), lambda r, t: (r, t)),
        ],
        out_shape=[
            jax.ShapeDtypeStruct((_B, _L), jnp.float32),
            jax.ShapeDtypeStruct((_B, _L), jnp.int32),
        ],
        compiler_params=pltpu.CompilerParams(
            dimension_semantics=("parallel", "arbitrary"),
        ),
    )(xcb, Wb)


_TK_BLK = 256


def _topk_body(bv_ref, ba_ref, w_ref, i_ref):
    work = bv_ref[...]
    lane = lax.broadcasted_iota(jnp.int32, (_TK_BLK, _L), 1)
    full_idx = (lane // 128) * _FB + (lane % 128) + ba_ref[...] * 128
    wcols = []
    icols = []
    neg_inf = jnp.float32(-jnp.inf)
    for _ in range(_K):
        m = jnp.max(work, axis=1, keepdims=True)
        at_max = work == m
        jm = jnp.min(jnp.where(at_max, lane, _L), axis=1, keepdims=True)
        sel = lane == jm
        idx = jnp.sum(jnp.where(sel, full_idx, 0), axis=1, keepdims=True)
        wcols.append(m)
        icols.append(idx)
        work = jnp.where(sel, neg_inf, work)
    w_ref[...] = jnp.concatenate(wcols, axis=1)
    i_ref[...] = jnp.concatenate(icols, axis=1)


def _topk(bv, ba):
    return pl.pallas_call(
        _topk_body,
        grid=(_B // _TK_BLK,),
        in_specs=[
            pl.BlockSpec((_TK_BLK, _L), lambda r: (r, 0)),
            pl.BlockSpec((_TK_BLK, _L), lambda r: (r, 0)),
        ],
        out_specs=[
            pl.BlockSpec((_TK_BLK, _K), lambda r: (r, 0)),
            pl.BlockSpec((_TK_BLK, _K), lambda r: (r, 0)),
        ],
        out_shape=[
            jax.ShapeDtypeStruct((_B, _K), jnp.float32),
            jax.ShapeDtypeStruct((_B, _K), jnp.int32),
        ],
        compiler_params=pltpu.CompilerParams(
            dimension_semantics=("parallel",),
        ),
    )(bv, ba)


_NW = 32
_RPW = _B // _NW
_GR = 8
_NBUF = 4


def _decode_body(wdec_hbm, w_hbm, idx_hbm, out_hbm, idx_v, w_v, rows_v, acc_v,
                 g0sem, g1sem, g2sem, g3sem, o0sem, o1sem):
    wid = lax.axis_index("c") * 16 + lax.axis_index("s")
    base = wid * _RPW
    pltpu.sync_copy(idx_hbm.at[pl.ds(base, _RPW)], idx_v)
    pltpu.sync_copy(w_hbm.at[pl.ds(base * _K, _RPW * _K)], w_v)
    gsems = (g0sem, g1sem, g2sem, g3sem)
    osems = (o0sem, o1sem)

    def gcopy(r, q, buf):
        return pltpu.make_async_copy(
            wdec_hbm.at[idx_v.at[r, pl.ds(q * _GR, _GR)]],
            rows_v.at[buf], gsems[buf])

    for q in range(4):
        gcopy(0, q, q).start()

    def pair_loop(r0, carry):
        for rr in range(2):
            r = r0 + rr
            for g in range(2):
                wvec = w_v[pl.ds(r * _K + g * 16, 16)]
                bw = [jnp.full((16,), wvec[j], jnp.float32)
                      for j in range(16)]
                gcopy(r, 2 * g, 2 * g).wait()
                gcopy(r, 2 * g + 1, 2 * g + 1).wait()

                if g == 0:
                    @pl.when(r >= 2)
                    def _(rr=rr, r=r):
                        pltpu.make_async_copy(
                            acc_v.at[rr], out_hbm.at[base + r - 2],
                            osems[rr]).wait()

                def chunk_loop(c, carry, g=g, rr=rr, bw=bw):
                    off = c * 16
                    if g == 0:
                        a = jnp.zeros((16,), jnp.float32)
                    else:
                        a = acc_v[rr, pl.ds(off, 16)]
                    for j in range(16):
                        a = a + bw[j] * \
                            rows_v[2 * g + j // _GR, j % _GR, pl.ds(off, 16)]
                    acc_v[rr, pl.ds(off, 16)] = a
                    return carry

                carry = lax.fori_loop(0, _D // 16, chunk_loop, carry)

                @pl.when(r + 1 < _RPW)
                def _(r=r, g=g):
                    gcopy(r + 1, 2 * g, 2 * g).start()
                    gcopy(r + 1, 2 * g + 1, 2 * g + 1).start()

            pltpu.make_async_copy(acc_v.at[rr], out_hbm.at[base + r],
                                  osems[rr]).start()
        return carry

    lax.fori_loop(0, _RPW // 2, lambda i, c: pair_loop(i * 2, c), 0)
    pltpu.make_async_copy(acc_v.at[0], out_hbm.at[base + _RPW - 2],
                          o0sem).wait()
    pltpu.make_async_copy(acc_v.at[1], out_hbm.at[base + _RPW - 1],
                          o1sem).wait()


def _decode(W_dec, weights, indices):
    mesh = plsc.VectorSubcoreMesh(core_axis_name="c", subcore_axis_name="s")
    run = pl.kernel(
        _decode_body,
        out_type=jax.ShapeDtypeStruct((_B, _D), jnp.float32),
        mesh=mesh,
        scratch_types=[
            pltpu.VMEM((_RPW, _K), jnp.int32),
            pltpu.VMEM((_RPW * _K,), jnp.float32),
            pltpu.VMEM((_NBUF, _GR, _D), jnp.float32),
            pltpu.VMEM((2, _D), jnp.float32),
            pltpu.SemaphoreType.DMA,
            pltpu.SemaphoreType.DMA,
            pltpu.SemaphoreType.DMA,
            pltpu.SemaphoreType.DMA,
            pltpu.SemaphoreType.DMA,
            pltpu.SemaphoreType.DMA,
        ],
    )
    return run(W_dec, weights.reshape(-1), indices)


_EP_BLK = 256


def _epilogue_body(x_ref, dec_ref, bp_ref, xn_ref, yn_ref, y_ref, rl_ref,
                   ls_ref):
    xn = x_ref[...] / _AVG_NORM * _SQRT_D
    yn = dec_ref[...] + bp_ref[...]
    xn_ref[...] = xn
    yn_ref[...] = yn
    y_ref[...] = yn * _AVG_NORM / _SQRT_D
    d = xn - yn
    rl = jnp.mean(d * d, axis=1)
    rl_ref[...] = rl
    ls_ref[...] = rl * (1.0 + _AUX_COEFF)


def _epilogue(x, decoded, b_post):
    return pl.pallas_call(
        _epilogue_body,
        grid=(_B // _EP_BLK,),
        in_specs=[
            pl.BlockSpec((_EP_BLK, _D), lambda r: (r, 0)),
            pl.BlockSpec((_EP_BLK, _D), lambda r: (r, 0)),
            pl.BlockSpec((1, _D), lambda r: (0, 0)),
        ],
        out_specs=[
            pl.BlockSpec((_EP_BLK, _D), lambda r: (r, 0)),
            pl.BlockSpec((_EP_BLK, _D), lambda r: (r, 0)),
            pl.BlockSpec((_EP_BLK, _D), lambda r: (r, 0)),
            pl.BlockSpec((_EP_BLK,), lambda r: (r,)),
            pl.BlockSpec((_EP_BLK,), lambda r: (r,)),
        ],
        out_shape=[
            jax.ShapeDtypeStruct((_B, _D), jnp.float32),
            jax.ShapeDtypeStruct((_B, _D), jnp.float32),
            jax.ShapeDtypeStruct((_B, _D), jnp.float32),
            jax.ShapeDtypeStruct((_B,), jnp.float32),
            jax.ShapeDtypeStruct((_B,), jnp.float32),
        ],
        compiler_params=pltpu.CompilerParams(
            dimension_semantics=("parallel",),
        ),
    )(x, decoded, b_post.reshape(1, _D))


def kernel(x, b_pre, W_enc, W_dec, b_post):
    xcb = (x / _AVG_NORM * _SQRT_D - b_pre[None, :]).astype(jnp.bfloat16)
    Wb = W_enc.astype(jnp.bfloat16)
    bv, ba = _encode(xcb, Wb)
    weights, indices = _topk(bv, ba)
    decoded = _decode(W_dec, weights, indices)
    x_normed, y_normed, y, recon_loss, loss = _epilogue(x, decoded, b_post)
    return (x_normed, x, weights, indices, y_normed, y, recon_loss,
            recon_loss, loss)


# speedup vs baseline: 1.5499x; 1.0547x over previous
import functools
import math

import jax
import jax.numpy as jnp
from jax import lax
from jax.experimental import pallas as pl
from jax.experimental.pallas import tpu as pltpu
from jax.experimental.pallas import tpu_sc as plsc

_D = 2048
_N = 32768
_K = 32
_B = 2048
_L = 1024
_FOLDS = _N // _L
_SQRT_D = math.sqrt(_D)
_AVG_NORM = 1.0
_AUX_COEFF = 1.0 / 32.0

_ROW_BLK = 512
_NRB = _B // _ROW_BLK
_FB = 4096
_NFB = _N // _FB


_FT = 2048
_MPT = _FT // 128


def _encode_body(x_ref, w_ref, bv_ref, ba_ref):
    t = pl.program_id(1)
    wb = w_ref[...].astype(jnp.bfloat16)
    acc = jnp.dot(x_ref[...], wb, preferred_element_type=jnp.float32)
    half = t % 2
    m0 = half * _MPT
    bvb = acc[:, 0:128]
    bab = jnp.full((_ROW_BLK, 128), m0, jnp.int32)
    for m in range(1, _MPT):
        sl = acc[:, m * 128:(m + 1) * 128]
        upd = sl >= bvb
        bvb = jnp.where(upd, sl, bvb)
        bab = jnp.where(upd, m0 + m, bab)

    @pl.when(half == 0)
    def _():
        bv_ref[...] = bvb
        ba_ref[...] = bab

    @pl.when(half == 1)
    def _():
        cur = bv_ref[...]
        upd = bvb >= cur
        bv_ref[...] = jnp.where(upd, bvb, cur)
        ba_ref[...] = jnp.where(upd, bab, ba_ref[...])


def _encode(xcb, W_enc):
    return pl.pallas_call(
        _encode_body,
        grid=(_NRB, _N // _FT),
        in_specs=[
            pl.BlockSpec((_ROW_BLK, _D), lambda r, t: (r, 0)),
            pl.BlockSpec((_D, _FT), lambda r, t: (0, t)),
        ],
        out_specs=[
            pl.BlockSpec((_ROW_BLK, 128), lambda r, t: (r, t // 2)),
            pl.BlockSpec((_ROW_BLK, 128), lambda r, t: (r, t // 2)),
        ],
        out_shape=[
            jax.ShapeDtypeStruct((_B, _L), jnp.float32),
            jax.ShapeDtypeStruct((_B, _L), jnp.int32),
        ],
        compiler_params=pltpu.CompilerParams(
            dimension_semantics=("parallel", "arbitrary"),
        ),
    )(xcb, W_enc)


_TK_BLK = 256


def _topk_body(bv_ref, ba_ref, w_ref, i_ref):
    work = bv_ref[...]
    lane = lax.broadcasted_iota(jnp.int32, (_TK_BLK, _L), 1)
    full_idx = (lane // 128) * _FB + (lane % 128) + ba_ref[...] * 128
    wcols = []
    icols = []
    neg_inf = jnp.float32(-jnp.inf)
    for _ in range(_K):
        m = jnp.max(work, axis=1, keepdims=True)
        at_max = work == m
        jm = jnp.min(jnp.where(at_max, lane, _L), axis=1, keepdims=True)
        sel = lane == jm
        idx = jnp.sum(jnp.where(sel, full_idx, 0), axis=1, keepdims=True)
        wcols.append(m)
        icols.append(idx)
        work = jnp.where(sel, neg_inf, work)
    w_ref[...] = jnp.concatenate(wcols, axis=1)
    i_ref[...] = jnp.concatenate(icols, axis=1)


def _topk(bv, ba):
    return pl.pallas_call(
        _topk_body,
        grid=(_B // _TK_BLK,),
        in_specs=[
            pl.BlockSpec((_TK_BLK, _L), lambda r: (r, 0)),
            pl.BlockSpec((_TK_BLK, _L), lambda r: (r, 0)),
        ],
        out_specs=[
            pl.BlockSpec((_TK_BLK, _K), lambda r: (r, 0)),
            pl.BlockSpec((_TK_BLK, _K), lambda r: (r, 0)),
        ],
        out_shape=[
            jax.ShapeDtypeStruct((_B, _K), jnp.float32),
            jax.ShapeDtypeStruct((_B, _K), jnp.int32),
        ],
        compiler_params=pltpu.CompilerParams(
            dimension_semantics=("parallel",),
        ),
    )(bv, ba)


_NW = 32
_RPW = _B // _NW
_GR = 8
_NBUF = 4


def _decode_body(wdec_hbm, w_hbm, idx_hbm, out_hbm, idx_v, w_v, rows_v, acc_v,
                 g0sem, g1sem, g2sem, g3sem, o0sem, o1sem):
    wid = lax.axis_index("c") * 16 + lax.axis_index("s")
    base = wid * _RPW
    pltpu.sync_copy(idx_hbm.at[pl.ds(base, _RPW)], idx_v)
    pltpu.sync_copy(w_hbm.at[pl.ds(base * _K, _RPW * _K)], w_v)
    gsems = (g0sem, g1sem, g2sem, g3sem)
    osems = (o0sem, o1sem)

    def gcopy(r, q, buf):
        return pltpu.make_async_copy(
            wdec_hbm.at[idx_v.at[r, pl.ds(q * _GR, _GR)]],
            rows_v.at[buf], gsems[buf])

    for q in range(4):
        gcopy(0, q, q).start()

    def pair_loop(r0, carry):
        for rr in range(2):
            r = r0 + rr
            for g in range(2):
                wvec = w_v[pl.ds(r * _K + g * 16, 16)]
                bw = [jnp.full((16,), wvec[j], jnp.float32)
                      for j in range(16)]
                gcopy(r, 2 * g, 2 * g).wait()
                gcopy(r, 2 * g + 1, 2 * g + 1).wait()

                if g == 0:
                    @pl.when(r >= 2)
                    def _(rr=rr, r=r):
                        pltpu.make_async_copy(
                            acc_v.at[rr], out_hbm.at[base + r - 2],
                            osems[rr]).wait()

                def chunk_loop(c, carry, g=g, rr=rr, bw=bw):
                    off = c * 16
                    if g == 0:
                        a = jnp.zeros((16,), jnp.float32)
                    else:
                        a = acc_v[rr, pl.ds(off, 16)]
                    for j in range(16):
                        a = a + bw[j] * \
                            rows_v[2 * g + j // _GR, j % _GR, pl.ds(off, 16)]
                    acc_v[rr, pl.ds(off, 16)] = a
                    return carry

                carry = lax.fori_loop(0, _D // 16, chunk_loop, carry)

                @pl.when(r + 1 < _RPW)
                def _(r=r, g=g):
                    gcopy(r + 1, 2 * g, 2 * g).start()
                    gcopy(r + 1, 2 * g + 1, 2 * g + 1).start()

            pltpu.make_async_copy(acc_v.at[rr], out_hbm.at[base + r],
                                  osems[rr]).start()
        return carry

    lax.fori_loop(0, _RPW // 2, lambda i, c: pair_loop(i * 2, c), 0)
    pltpu.make_async_copy(acc_v.at[0], out_hbm.at[base + _RPW - 2],
                          o0sem).wait()
    pltpu.make_async_copy(acc_v.at[1], out_hbm.at[base + _RPW - 1],
                          o1sem).wait()


def _decode(W_dec, weights, indices):
    mesh = plsc.VectorSubcoreMesh(core_axis_name="c", subcore_axis_name="s")
    run = pl.kernel(
        _decode_body,
        out_type=jax.ShapeDtypeStruct((_B, _D), jnp.float32),
        mesh=mesh,
        scratch_types=[
            pltpu.VMEM((_RPW, _K), jnp.int32),
            pltpu.VMEM((_RPW * _K,), jnp.float32),
            pltpu.VMEM((_NBUF, _GR, _D), jnp.float32),
            pltpu.VMEM((2, _D), jnp.float32),
            pltpu.SemaphoreType.DMA,
            pltpu.SemaphoreType.DMA,
            pltpu.SemaphoreType.DMA,
            pltpu.SemaphoreType.DMA,
            pltpu.SemaphoreType.DMA,
            pltpu.SemaphoreType.DMA,
        ],
    )
    return run(W_dec, weights.reshape(-1), indices)


_EP_BLK = 256


def _epilogue_body(x_ref, dec_ref, bp_ref, xn_ref, yn_ref, y_ref, rl_ref,
                   ls_ref):
    xn = x_ref[...] / _AVG_NORM * _SQRT_D
    yn = dec_ref[...] + bp_ref[...]
    xn_ref[...] = xn
    yn_ref[...] = yn
    y_ref[...] = yn * _AVG_NORM / _SQRT_D
    d = xn - yn
    rl = jnp.mean(d * d, axis=1)
    rl_ref[...] = rl
    ls_ref[...] = rl * (1.0 + _AUX_COEFF)


def _epilogue(x, decoded, b_post):
    return pl.pallas_call(
        _epilogue_body,
        grid=(_B // _EP_BLK,),
        in_specs=[
            pl.BlockSpec((_EP_BLK, _D), lambda r: (r, 0)),
            pl.BlockSpec((_EP_BLK, _D), lambda r: (r, 0)),
            pl.BlockSpec((1, _D), lambda r: (0, 0)),
        ],
        out_specs=[
            pl.BlockSpec((_EP_BLK, _D), lambda r: (r, 0)),
            pl.BlockSpec((_EP_BLK, _D), lambda r: (r, 0)),
            pl.BlockSpec((_EP_BLK, _D), lambda r: (r, 0)),
            pl.BlockSpec((_EP_BLK,), lambda r: (r,)),
            pl.BlockSpec((_EP_BLK,), lambda r: (r,)),
        ],
        out_shape=[
            jax.ShapeDtypeStruct((_B, _D), jnp.float32),
            jax.ShapeDtypeStruct((_B, _D), jnp.float32),
            jax.ShapeDtypeStruct((_B, _D), jnp.float32),
            jax.ShapeDtypeStruct((_B,), jnp.float32),
            jax.ShapeDtypeStruct((_B,), jnp.float32),
        ],
        compiler_params=pltpu.CompilerParams(
            dimension_semantics=("parallel",),
        ),
    )(x, decoded, b_post.reshape(1, _D))


def kernel(x, b_pre, W_enc, W_dec, b_post):
    xcb = (x / _AVG_NORM * _SQRT_D - b_pre[None, :]).astype(jnp.bfloat16)
    bv, ba = _encode(xcb, W_enc)
    weights, indices = _topk(bv, ba)
    decoded = _decode(W_dec, weights, indices)
    x_normed, y_normed, y, recon_loss, loss = _epilogue(x, decoded, b_post)
    return (x_normed, x, weights, indices, y_normed, y, recon_loss,
            recon_loss, loss)


# speedup vs baseline: 1.5530x; 1.0020x over previous
import functools
import math

import jax
import jax.numpy as jnp
from jax import lax
from jax.experimental import pallas as pl
from jax.experimental.pallas import tpu as pltpu
from jax.experimental.pallas import tpu_sc as plsc

_D = 2048
_N = 32768
_K = 32
_B = 2048
_L = 1024
_FOLDS = _N // _L
_SQRT_D = math.sqrt(_D)
_AVG_NORM = 1.0
_AUX_COEFF = 1.0 / 32.0

_ROW_BLK = 512
_NRB = _B // _ROW_BLK
_FB = 4096
_NFB = _N // _FB


_FT = 2048
_MPT = _FT // 128


def _encode_body(x_ref, w_ref, bv_ref, ba_ref):
    t = pl.program_id(1)
    wb = w_ref[...].astype(jnp.bfloat16)
    acc = jnp.dot(x_ref[...], wb, preferred_element_type=jnp.float32)
    half = t % 2
    m0 = half * _MPT
    bvb = acc[:, 0:128]
    bab = jnp.full((_ROW_BLK, 128), m0, jnp.int32)
    for m in range(1, _MPT):
        sl = acc[:, m * 128:(m + 1) * 128]
        upd = sl >= bvb
        bvb = jnp.where(upd, sl, bvb)
        bab = jnp.where(upd, m0 + m, bab)

    @pl.when(half == 0)
    def _():
        bv_ref[...] = bvb
        ba_ref[...] = bab

    @pl.when(half == 1)
    def _():
        cur = bv_ref[...]
        upd = bvb >= cur
        bv_ref[...] = jnp.where(upd, bvb, cur)
        ba_ref[...] = jnp.where(upd, bab, ba_ref[...])


def _encode(xcb, W_enc):
    return pl.pallas_call(
        _encode_body,
        grid=(_NRB, _N // _FT),
        in_specs=[
            pl.BlockSpec((_ROW_BLK, _D), lambda r, t: (r, 0)),
            pl.BlockSpec((_D, _FT), lambda r, t: (0, t)),
        ],
        out_specs=[
            pl.BlockSpec((_ROW_BLK, 128), lambda r, t: (r, t // 2)),
            pl.BlockSpec((_ROW_BLK, 128), lambda r, t: (r, t // 2)),
        ],
        out_shape=[
            jax.ShapeDtypeStruct((_B, _L), jnp.float32),
            jax.ShapeDtypeStruct((_B, _L), jnp.int32),
        ],
        compiler_params=pltpu.CompilerParams(
            dimension_semantics=("parallel", "arbitrary"),
        ),
    )(xcb, W_enc)


_TK_BLK = 256


def _topk_body(bv_ref, ba_ref, w_ref, i_ref):
    work = bv_ref[...]
    lane = lax.broadcasted_iota(jnp.int32, (_TK_BLK, _L), 1)
    full_idx = (lane // 128) * _FB + (lane % 128) + ba_ref[...] * 128
    wcols = []
    icols = []
    neg_inf = jnp.float32(-jnp.inf)
    for _ in range(_K):
        m = jnp.max(work, axis=1, keepdims=True)
        at_max = work == m
        jm = jnp.min(jnp.where(at_max, lane, _L), axis=1, keepdims=True)
        sel = lane == jm
        idx = jnp.sum(jnp.where(sel, full_idx, 0), axis=1, keepdims=True)
        wcols.append(m)
        icols.append(idx)
        work = jnp.where(sel, neg_inf, work)
    w_ref[...] = jnp.concatenate(wcols, axis=1)
    i_ref[...] = jnp.concatenate(icols, axis=1)


def _topk(bv, ba):
    return pl.pallas_call(
        _topk_body,
        grid=(_B // _TK_BLK,),
        in_specs=[
            pl.BlockSpec((_TK_BLK, _L), lambda r: (r, 0)),
            pl.BlockSpec((_TK_BLK, _L), lambda r: (r, 0)),
        ],
        out_specs=[
            pl.BlockSpec((_TK_BLK, _K), lambda r: (r, 0)),
            pl.BlockSpec((_TK_BLK, _K), lambda r: (r, 0)),
        ],
        out_shape=[
            jax.ShapeDtypeStruct((_B, _K), jnp.float32),
            jax.ShapeDtypeStruct((_B, _K), jnp.int32),
        ],
        compiler_params=pltpu.CompilerParams(
            dimension_semantics=("parallel",),
        ),
    )(bv, ba)


_NW = 32
_RPW = _B // _NW
_GR = 8
_NBUF = 4


def _decode_body(wdec_hbm, w_hbm, idx_hbm, out_hbm, idx_v, w_v, rows_v, acc_v,
                 g0sem, g1sem, g2sem, g3sem, o0sem, o1sem):
    wid = lax.axis_index("c") * 16 + lax.axis_index("s")
    base = wid * _RPW
    pltpu.sync_copy(idx_hbm.at[pl.ds(base, _RPW)], idx_v)
    pltpu.sync_copy(w_hbm.at[pl.ds(base * _K, _RPW * _K)], w_v)
    gsems = (g0sem, g1sem, g2sem, g3sem)
    osems = (o0sem, o1sem)

    def gcopy(r, q, buf):
        return pltpu.make_async_copy(
            wdec_hbm.at[idx_v.at[r, pl.ds(q * _GR, _GR)]],
            rows_v.at[buf], gsems[buf])

    for q in range(4):
        gcopy(0, q, q).start()

    def pair_loop(r0, carry):
        for rr in range(2):
            r = r0 + rr
            for g in range(2):
                wvec = w_v[pl.ds(r * _K + g * 16, 16)]
                bw = [jnp.full((16,), wvec[j], jnp.float32)
                      for j in range(16)]
                gcopy(r, 2 * g, 2 * g).wait()
                gcopy(r, 2 * g + 1, 2 * g + 1).wait()

                if g == 0:
                    @pl.when(r >= 2)
                    def _(rr=rr, r=r):
                        pltpu.make_async_copy(
                            acc_v.at[rr], out_hbm.at[base + r - 2],
                            osems[rr]).wait()

                def chunk_loop(c, carry, g=g, rr=rr, bw=bw):
                    for u in range(2):
                        off = c * 32 + u * 16
                        if g == 0:
                            a = jnp.zeros((16,), jnp.float32)
                        else:
                            a = acc_v[rr, pl.ds(off, 16)]
                        for j in range(16):
                            a = a + bw[j] * \
                                rows_v[2 * g + j // _GR, j % _GR,
                                       pl.ds(off, 16)]
                        acc_v[rr, pl.ds(off, 16)] = a
                    return carry

                carry = lax.fori_loop(0, _D // 32, chunk_loop, carry)

                @pl.when(r + 1 < _RPW)
                def _(r=r, g=g):
                    gcopy(r + 1, 2 * g, 2 * g).start()
                    gcopy(r + 1, 2 * g + 1, 2 * g + 1).start()

            pltpu.make_async_copy(acc_v.at[rr], out_hbm.at[base + r],
                                  osems[rr]).start()
        return carry

    lax.fori_loop(0, _RPW // 2, lambda i, c: pair_loop(i * 2, c), 0)
    pltpu.make_async_copy(acc_v.at[0], out_hbm.at[base + _RPW - 2],
                          o0sem).wait()
    pltpu.make_async_copy(acc_v.at[1], out_hbm.at[base + _RPW - 1],
                          o1sem).wait()


def _decode(W_dec, weights, indices):
    mesh = plsc.VectorSubcoreMesh(core_axis_name="c", subcore_axis_name="s")
    run = pl.kernel(
        _decode_body,
        out_type=jax.ShapeDtypeStruct((_B, _D), jnp.float32),
        mesh=mesh,
        scratch_types=[
            pltpu.VMEM((_RPW, _K), jnp.int32),
            pltpu.VMEM((_RPW * _K,), jnp.float32),
            pltpu.VMEM((_NBUF, _GR, _D), jnp.float32),
            pltpu.VMEM((2, _D), jnp.float32),
            pltpu.SemaphoreType.DMA,
            pltpu.SemaphoreType.DMA,
            pltpu.SemaphoreType.DMA,
            pltpu.SemaphoreType.DMA,
            pltpu.SemaphoreType.DMA,
            pltpu.SemaphoreType.DMA,
        ],
    )
    return run(W_dec, weights.reshape(-1), indices)


_EP_BLK = 256


def _epilogue_body(x_ref, dec_ref, bp_ref, xn_ref, yn_ref, y_ref, rl_ref,
                   ls_ref):
    xn = x_ref[...] / _AVG_NORM * _SQRT_D
    yn = dec_ref[...] + bp_ref[...]
    xn_ref[...] = xn
    yn_ref[...] = yn
    y_ref[...] = yn * _AVG_NORM / _SQRT_D
    d = xn - yn
    rl = jnp.mean(d * d, axis=1)
    rl_ref[...] = rl
    ls_ref[...] = rl * (1.0 + _AUX_COEFF)


def _epilogue(x, decoded, b_post):
    return pl.pallas_call(
        _epilogue_body,
        grid=(_B // _EP_BLK,),
        in_specs=[
            pl.BlockSpec((_EP_BLK, _D), lambda r: (r, 0)),
            pl.BlockSpec((_EP_BLK, _D), lambda r: (r, 0)),
            pl.BlockSpec((1, _D), lambda r: (0, 0)),
        ],
        out_specs=[
            pl.BlockSpec((_EP_BLK, _D), lambda r: (r, 0)),
            pl.BlockSpec((_EP_BLK, _D), lambda r: (r, 0)),
            pl.BlockSpec((_EP_BLK, _D), lambda r: (r, 0)),
            pl.BlockSpec((_EP_BLK,), lambda r: (r,)),
            pl.BlockSpec((_EP_BLK,), lambda r: (r,)),
        ],
        out_shape=[
            jax.ShapeDtypeStruct((_B, _D), jnp.float32),
            jax.ShapeDtypeStruct((_B, _D), jnp.float32),
            jax.ShapeDtypeStruct((_B, _D), jnp.float32),
            jax.ShapeDtypeStruct((_B,), jnp.float32),
            jax.ShapeDtypeStruct((_B,), jnp.float32),
        ],
        compiler_params=pltpu.CompilerParams(
            dimension_semantics=("parallel",),
        ),
    )(x, decoded, b_post.reshape(1, _D))


def kernel(x, b_pre, W_enc, W_dec, b_post):
    xcb = (x / _AVG_NORM * _SQRT_D - b_pre[None, :]).astype(jnp.bfloat16)
    bv, ba = _encode(xcb, W_enc)
    weights, indices = _topk(bv, ba)
    decoded = _decode(W_dec, weights, indices)
    x_normed, y_normed, y, recon_loss, loss = _epilogue(x, decoded, b_post)
    return (x_normed, x, weights, indices, y_normed, y, recon_loss,
            recon_loss, loss)
